# Initial kernel scaffold; baseline (speedup 1.0000x reference)
import jax
import jax.numpy as jnp
from jax.experimental import pallas as pl


def kernel(species, embedding, idx_0, idx_1, idx_2, idx_3, idx_4, idx_5, idx_6, idx_7, W1, b1, W2, b2):
    raise NotImplementedError("write your pallas kernel here")


# speedup vs baseline: 6.6714x; 6.6714x over previous
import functools

import jax
import jax.numpy as jnp
import numpy as np
from jax.experimental import pallas as pl
from jax.experimental.pallas import tpu as pltpu
from jax.experimental.pallas import tpu_sc as plsc

N = 32768
D = 768
H = 64
O = 16
E = 8
PER = N // E

TILE = 1024
SC_WIN = 256


def _sc_route_mask(onehot_src, idx_cat):
    idx2 = idx_cat.reshape(1, N)
    steps_per_block = PER // SC_WIN
    mesh = plsc.VectorSubcoreMesh(core_axis_name="core", subcore_axis_name="subcore")

    @functools.partial(
        pl.kernel,
        out_type=jax.ShapeDtypeStruct((N, 128), jnp.float32),
        mesh=mesh,
    )
    def sc_kernel(src_hbm, i_hbm, o_hbm):
        def body(src_vmem, i_vmem):
            pltpu.sync_copy(src_vmem, o_hbm.at[i_vmem.at[0]])

        pltpu.emit_pipeline(
            body,
            grid=(N // SC_WIN,),
            in_specs=[
                pl.BlockSpec((SC_WIN, 128), lambda i: (i // steps_per_block, 0)),
                pl.BlockSpec((1, SC_WIN), lambda i: (0, i)),
            ],
            out_specs=[],
            core_axis_name=("core", "subcore"),
            dimension_semantics=(pltpu.PARALLEL,),
        )(src_hbm, i_hbm)

    return sc_kernel(onehot_src, idx2)


def _mlp_kernel(emb_ref, mask_ref, w1_ref, b1_ref, w2_ref, b2_ref, out_ref):
    x = emb_ref[...].astype(jnp.bfloat16)
    h = jax.lax.dot_general(
        x, w1_ref[...], (((1,), (0,)), ((), ())),
        preferred_element_type=jnp.float32,
    ) + b1_ref[...]
    h = h * jax.nn.sigmoid(h)
    o_all = jax.lax.dot_general(
        h.astype(jnp.bfloat16), w2_ref[...], (((1,), (0,)), ((), ())),
        preferred_element_type=jnp.float32,
    ) + b2_ref[...]
    mrep = mask_ref[...].astype(jnp.float32)
    om = o_all * mrep
    acc = om[:, 0:O]
    for e in range(1, E):
        acc = acc + om[:, e * O:(e + 1) * O]
    out_ref[...] = acc


def kernel(species, embedding, idx_0, idx_1, idx_2, idx_3, idx_4, idx_5,
           idx_6, idx_7, W1, b1, W2, b2):
    idx_cat = jnp.concatenate(
        [idx_0, idx_1, idx_2, idx_3, idx_4, idx_5, idx_6, idx_7])

    onehot_src = jnp.broadcast_to(
        jnp.repeat(jnp.eye(E, dtype=jnp.float32), 16, axis=1)[:, None, :],
        (E, SC_WIN, 128),
    ).reshape(E * SC_WIN, 128)

    mask = _sc_route_mask(onehot_src, idx_cat)

    w1_cat = jnp.transpose(W1, (1, 0, 2)).reshape(D, E * H).astype(jnp.bfloat16)
    b1_cat = b1.reshape(1, E * H)
    w2_big = jnp.zeros((E, H, E, O), jnp.float32)
    w2_big = w2_big.at[jnp.arange(E), :, jnp.arange(E), :].set(W2)
    w2_big = w2_big.reshape(E * H, E * O).astype(jnp.bfloat16)
    b2_big = b2.reshape(1, E * O)

    out = pl.pallas_call(
        _mlp_kernel,
        grid=(N // TILE,),
        in_specs=[
            pl.BlockSpec((TILE, D), lambda i: (i, 0)),
            pl.BlockSpec((TILE, 128), lambda i: (i, 0)),
            pl.BlockSpec((D, E * H), lambda i: (0, 0)),
            pl.BlockSpec((1, E * H), lambda i: (0, 0)),
            pl.BlockSpec((E * H, E * O), lambda i: (0, 0)),
            pl.BlockSpec((1, E * O), lambda i: (0, 0)),
        ],
        out_specs=pl.BlockSpec((TILE, O), lambda i: (i, 0)),
        out_shape=jax.ShapeDtypeStruct((N, O), jnp.float32),
        compiler_params=pltpu.CompilerParams(
            dimension_semantics=("arbitrary",),
        ),
    )(embedding, mask, w1_cat, b1_cat, w2_big, b2_big)
    return out
